# baseline (speedup 1.0000x reference)
import functools

import jax
import jax.numpy as jnp
from jax import lax
from jax.experimental import pallas as pl
from jax.experimental.pallas import tpu as pltpu
from jax.experimental.pallas import tpu_sc as plsc

BATCH = 16384
EMBED_DIM = 32
_ROWS8 = 125001

_NC = 2
_NS = 16
_NW = _NC * _NS
_BPW = BATCH // _NW
_CHUNK = 128
_NCHUNK = _BPW // _CHUNK
_NCG = BATCH // _CHUNK


def _prep_idx(idx_v, qcall):
    for g in range(_BPW // 16):
        v = idx_v[pl.ds(g * 16, 16)]
        q = lax.shift_right_logical(v, 3)
        for c in range(EMBED_DIM):
            qcall[c, pl.ds(g * 16, 16)] = q + (c * _ROWS8)


def _gather_one_table(tab2, qcall, dst, out, c0, wid, gsem, wsem):
    def dslice(buf, c):
        return dst.at[pl.ds((buf * EMBED_DIM + c) * _CHUNK, _CHUNK)]

    def fire(ch, buf):
        for c in range(EMBED_DIM):
            pltpu.async_copy(
                tab2.at[qcall.at[c, pl.ds(ch * _CHUNK, _CHUNK)]],
                dslice(buf, c), gsem[buf])

    def drain_gather(buf):
        for c in range(EMBED_DIM):
            pltpu.make_async_copy(
                tab2.at[qcall.at[0, pl.ds(0, _CHUNK)]],
                dslice(buf, c), gsem[buf]).wait()

    def write_out(ch, buf):
        chg = wid * _NCHUNK + ch
        for c in range(EMBED_DIM):
            pltpu.async_copy(dslice(buf, c), out.at[c0 + c, chg], wsem[buf])

    def drain_writes(buf):
        for c in range(EMBED_DIM):
            pltpu.make_async_copy(dslice(buf, c), out.at[c0, 0],
                                  wsem[buf]).wait()

    fire(0, 0)
    for ch in range(_NCHUNK):
        buf = ch % 2
        if ch + 1 < _NCHUNK:
            if ch >= 1:
                drain_writes((ch + 1) % 2)
            fire(ch + 1, (ch + 1) % 2)
        drain_gather(buf)
        write_out(ch, buf)
    drain_writes(0)
    drain_writes(1)


def _gather_body(uidx_hbm, iidx_hbm, utab2, itab2, out,
                 idx_v, qcall, dst, gsem0, gsem1, wsem0, wsem1):
    c = lax.axis_index("c")
    s = lax.axis_index("s")
    wid = s * _NC + c
    base = wid * _BPW
    gsem = (gsem0, gsem1)
    wsem = (wsem0, wsem1)

    pltpu.sync_copy(uidx_hbm.at[pl.ds(base, _BPW)], idx_v)
    _prep_idx(idx_v, qcall)
    _gather_one_table(utab2, qcall, dst, out, 0, wid, gsem, wsem)

    pltpu.sync_copy(iidx_hbm.at[pl.ds(base, _BPW)], idx_v)
    _prep_idx(idx_v, qcall)
    _gather_one_table(itab2, qcall, dst, out, EMBED_DIM, wid, gsem, wsem)


@jax.jit
def _sc_gather(uidx, iidx, utab2, itab2):
    mesh = plsc.VectorSubcoreMesh(core_axis_name="c", subcore_axis_name="s")
    f = functools.partial(
        pl.kernel,
        mesh=mesh,
        out_type=jax.ShapeDtypeStruct((2 * EMBED_DIM, _NCG, _CHUNK, 8),
                                      jnp.float32),
        scratch_types=[
            pltpu.VMEM((_BPW,), jnp.int32),
            pltpu.VMEM((EMBED_DIM, _BPW), jnp.int32),
            pltpu.VMEM((2 * EMBED_DIM * _CHUNK, 8), jnp.float32),
            pltpu.SemaphoreType.DMA,
            pltpu.SemaphoreType.DMA,
            pltpu.SemaphoreType.DMA,
            pltpu.SemaphoreType.DMA,
        ],
        compiler_params=pltpu.CompilerParams(use_tc_tiling_on_sc=False),
    )(_gather_body)
    return f(uidx, iidx, utab2, itab2)


_BB = 1024
_NB = BATCH // _BB
_FB = _BB * 8


def _mlp_body(x_ref, m_ref, w1u_ref, w1i_ref, b1_ref, w2_ref, b2_ref,
              w3_ref, b3_ref, out_ref):
    xu = x_ref[pl.ds(0, EMBED_DIM), :].reshape(EMBED_DIM, _BB, 8)
    xi = x_ref[pl.ds(EMBED_DIM, EMBED_DIM), :].reshape(EMBED_DIM, _BB, 8)
    mu = m_ref[0:1, :].reshape(1, _BB, 8)
    mi = m_ref[1:2, :].reshape(1, _BB, 8)
    x_u = jnp.sum(xu * mu, axis=2)
    x_i = jnp.sum(xi * mi, axis=2)
    h1 = jnp.dot(w1u_ref[...], x_u, preferred_element_type=jnp.float32)
    h1 = h1 + jnp.dot(w1i_ref[...], x_i, preferred_element_type=jnp.float32)
    h1 = jnp.maximum(h1 + b1_ref[...], 0.0)
    h2 = jnp.dot(w2_ref[...], h1, preferred_element_type=jnp.float32)
    h2 = jnp.maximum(h2 + b2_ref[...], 0.0)
    o = jnp.sum(h2 * w3_ref[...], axis=0) + b3_ref[0, 0]
    out_ref[0, :] = o


@jax.jit
def _tc_mlp(xfat, msel, w1u, w1i, b1, w2, b2, w3, b3):
    out2d = pl.pallas_call(
        _mlp_body,
        grid=(_NB,),
        in_specs=[
            pl.BlockSpec((2 * EMBED_DIM, _FB), lambda b: (0, b)),
            pl.BlockSpec((2, _FB), lambda b: (0, b)),
            pl.BlockSpec((64, EMBED_DIM), lambda b: (0, 0)),
            pl.BlockSpec((64, EMBED_DIM), lambda b: (0, 0)),
            pl.BlockSpec((64, 1), lambda b: (0, 0)),
            pl.BlockSpec((32, 64), lambda b: (0, 0)),
            pl.BlockSpec((32, 1), lambda b: (0, 0)),
            pl.BlockSpec((32, 1), lambda b: (0, 0)),
            pl.BlockSpec((1, 1), lambda b: (0, 0)),
        ],
        out_specs=pl.BlockSpec((1, _BB), lambda b: (0, b)),
        out_shape=jax.ShapeDtypeStruct((1, BATCH), jnp.float32),
    )(xfat, msel, w1u, w1i, b1, w2, b2, w3, b3)
    return out2d.reshape(BATCH)


def kernel(user, item, user_table, item_table, W1, b1, W2, b2, W3, b3):
    uidx = user.astype(jnp.int32)
    iidx = item.astype(jnp.int32)
    utab2 = jnp.pad(user_table.T, ((0, 0), (0, 7))).reshape(
        EMBED_DIM * _ROWS8, 8)
    itab2 = jnp.pad(item_table.T, ((0, 0), (0, 7))).reshape(
        EMBED_DIM * _ROWS8, 8)
    xfat4 = _sc_gather(uidx, iidx, utab2, itab2)
    xfat = xfat4.reshape(2 * EMBED_DIM, BATCH * 8)
    k8 = jnp.arange(8, dtype=jnp.int32)[None, :]
    mu = (lax.bitwise_and(uidx, 7)[:, None] == k8).astype(jnp.float32)
    mi = (lax.bitwise_and(iidx, 7)[:, None] == k8).astype(jnp.float32)
    msel = jnp.stack([mu.reshape(-1), mi.reshape(-1)], axis=0)
    return _tc_mlp(xfat, msel, W1[:, :EMBED_DIM], W1[:, EMBED_DIM:],
                   b1.reshape(64, 1), W2, b2.reshape(32, 1),
                   W3.reshape(32, 1), b3.reshape(1, 1))


# speedup vs baseline: 1.0552x; 1.0552x over previous
import functools

import jax
import jax.numpy as jnp
from jax import lax
from jax.experimental import pallas as pl
from jax.experimental.pallas import tpu as pltpu
from jax.experimental.pallas import tpu_sc as plsc

BATCH = 16384
EMBED_DIM = 32

_NC = 2
_NS = 16
_NW = _NC * _NS
_BPW = BATCH // _NW
_HALF = _BPW // 2
_GRP = 16


def _gather_one_table(tab, idx_v, qbuf, ubuf, out, c0, base, sem):
    for g in range(_BPW // 16):
        v = idx_v[pl.ds(g * 16, 16)]
        qbuf[pl.ds(g * 16, 16)] = lax.bitwise_and(v, -8)

    for half in range(2):
        def group(g, _):
            j0 = g * _GRP
            qvec = qbuf[pl.ds(half * _HALF + j0, _GRP)]
            copies = []
            for k in range(_GRP):
                qa = pl.multiple_of(qvec[k], 8)
                copies.append(pltpu.async_copy(
                    tab.at[:, pl.ds(qa, 8)],
                    ubuf.at[:, pl.ds(pl.multiple_of((j0 + k) * 8, 8), 8)],
                    sem))
            for cp in copies:
                cp.wait()
            return 0

        lax.fori_loop(0, _HALF // _GRP, group, 0)
        pltpu.sync_copy(
            ubuf, out.at[pl.ds(c0, EMBED_DIM),
                         pl.ds((base + half * _HALF) * 8, _HALF * 8)])


def _gather_body(uidx_hbm, iidx_hbm, utab, itab, out,
                 idx_v, qbuf, ubuf, sem):
    c = lax.axis_index("c")
    s = lax.axis_index("s")
    wid = s * _NC + c
    base = wid * _BPW

    pltpu.sync_copy(uidx_hbm.at[pl.ds(base, _BPW)], idx_v)
    _gather_one_table(utab, idx_v, qbuf, ubuf, out, 0, base, sem)

    pltpu.sync_copy(iidx_hbm.at[pl.ds(base, _BPW)], idx_v)
    _gather_one_table(itab, idx_v, qbuf, ubuf, out, EMBED_DIM, base, sem)


@jax.jit
def _sc_gather(uidx, iidx, utab, itab):
    mesh = plsc.VectorSubcoreMesh(core_axis_name="c", subcore_axis_name="s")
    f = functools.partial(
        pl.kernel,
        mesh=mesh,
        out_type=jax.ShapeDtypeStruct((2 * EMBED_DIM, BATCH * 8),
                                      jnp.float32),
        scratch_types=[
            pltpu.VMEM((_BPW,), jnp.int32),
            pltpu.VMEM((_BPW,), jnp.int32),
            pltpu.VMEM((EMBED_DIM, _HALF * 8), jnp.float32),
            pltpu.SemaphoreType.DMA,
        ],
        compiler_params=pltpu.CompilerParams(use_tc_tiling_on_sc=False),
    )(_gather_body)
    return f(uidx, iidx, utab, itab)


_BB = 1024
_NB = BATCH // _BB
_FB = _BB * 8


def _mlp_body(x_ref, m_ref, w1u_ref, w1i_ref, b1_ref, w2_ref, b2_ref,
              w3_ref, b3_ref, out_ref):
    xu = x_ref[pl.ds(0, EMBED_DIM), :].reshape(EMBED_DIM, _BB, 8)
    xi = x_ref[pl.ds(EMBED_DIM, EMBED_DIM), :].reshape(EMBED_DIM, _BB, 8)
    mu = m_ref[0:1, :].reshape(1, _BB, 8)
    mi = m_ref[1:2, :].reshape(1, _BB, 8)
    x_u = jnp.sum(xu * mu, axis=2)
    x_i = jnp.sum(xi * mi, axis=2)
    h1 = jnp.dot(w1u_ref[...], x_u, preferred_element_type=jnp.float32)
    h1 = h1 + jnp.dot(w1i_ref[...], x_i, preferred_element_type=jnp.float32)
    h1 = jnp.maximum(h1 + b1_ref[...], 0.0)
    h2 = jnp.dot(w2_ref[...], h1, preferred_element_type=jnp.float32)
    h2 = jnp.maximum(h2 + b2_ref[...], 0.0)
    o = jnp.sum(h2 * w3_ref[...], axis=0) + b3_ref[0, 0]
    out_ref[0, :] = o


@jax.jit
def _tc_mlp(xfat, msel, w1u, w1i, b1, w2, b2, w3, b3):
    out2d = pl.pallas_call(
        _mlp_body,
        grid=(_NB,),
        in_specs=[
            pl.BlockSpec((2 * EMBED_DIM, _FB), lambda b: (0, b)),
            pl.BlockSpec((2, _FB), lambda b: (0, b)),
            pl.BlockSpec((64, EMBED_DIM), lambda b: (0, 0)),
            pl.BlockSpec((64, EMBED_DIM), lambda b: (0, 0)),
            pl.BlockSpec((64, 1), lambda b: (0, 0)),
            pl.BlockSpec((32, 64), lambda b: (0, 0)),
            pl.BlockSpec((32, 1), lambda b: (0, 0)),
            pl.BlockSpec((32, 1), lambda b: (0, 0)),
            pl.BlockSpec((1, 1), lambda b: (0, 0)),
        ],
        out_specs=pl.BlockSpec((1, _BB), lambda b: (0, b)),
        out_shape=jax.ShapeDtypeStruct((1, BATCH), jnp.float32),
    )(xfat, msel, w1u, w1i, b1, w2, b2, w3, b3)
    return out2d.reshape(BATCH)


def kernel(user, item, user_table, item_table, W1, b1, W2, b2, W3, b3):
    uidx = user.astype(jnp.int32)
    iidx = item.astype(jnp.int32)
    xfat = _sc_gather(uidx, iidx, user_table.T, item_table.T)
    k8 = jnp.arange(8, dtype=jnp.int32)[None, :]
    mu = (lax.bitwise_and(uidx, 7)[:, None] == k8).astype(jnp.float32)
    mi = (lax.bitwise_and(iidx, 7)[:, None] == k8).astype(jnp.float32)
    msel = jnp.stack([mu.reshape(-1), mi.reshape(-1)], axis=0)
    return _tc_mlp(xfat, msel, W1[:, :EMBED_DIM], W1[:, EMBED_DIM:],
                   b1.reshape(64, 1), W2, b2.reshape(32, 1),
                   W3.reshape(32, 1), b3.reshape(1, 1))


# speedup vs baseline: 1.1475x; 1.0875x over previous
import functools

import jax
import jax.numpy as jnp
from jax import lax
from jax.experimental import pallas as pl
from jax.experimental.pallas import tpu as pltpu
from jax.experimental.pallas import tpu_sc as plsc

BATCH = 16384
EMBED_DIM = 32
_PADR = 1000008

_NC = 2
_NS = 16
_NW = _NC * _NS
_BPW = BATCH // _NW
_CHUNK = 128
_NCHUNK = _BPW // _CHUNK


def _prep_idx(idx_v, qcall):
    for g in range(_BPW // 16):
        v = idx_v[pl.ds(g * 16, 16)]
        for c in range(EMBED_DIM):
            qcall[c, pl.ds(g * 16, 16)] = v + (c * _PADR)


def _gather_one_table(tab1, qcall, dst, out, c0, wid, gsem, wsem):
    def dslice(buf, c):
        return dst.at[pl.ds((buf * EMBED_DIM + c) * _CHUNK, _CHUNK)]

    def fire(ch, buf):
        for c in range(EMBED_DIM):
            pltpu.async_copy(
                tab1.at[qcall.at[c, pl.ds(ch * _CHUNK, _CHUNK)]],
                dslice(buf, c), gsem[buf])

    def drain_gather(buf):
        for c in range(EMBED_DIM):
            pltpu.make_async_copy(
                tab1.at[qcall.at[0, pl.ds(0, _CHUNK)]],
                dslice(buf, c), gsem[buf]).wait()

    def write_out(ch, buf):
        chg = wid * _NCHUNK + ch
        for c in range(EMBED_DIM):
            pltpu.async_copy(
                dslice(buf, c),
                out.at[c0 + c, pl.ds(chg * _CHUNK, _CHUNK)], wsem[buf])

    def drain_writes(buf):
        for c in range(EMBED_DIM):
            pltpu.make_async_copy(
                dslice(buf, c), out.at[c0, pl.ds(0, _CHUNK)],
                wsem[buf]).wait()

    fire(0, 0)
    for ch in range(_NCHUNK):
        buf = ch % 2
        if ch + 1 < _NCHUNK:
            if ch >= 1:
                drain_writes((ch + 1) % 2)
            fire(ch + 1, (ch + 1) % 2)
        drain_gather(buf)
        write_out(ch, buf)
    drain_writes(0)
    drain_writes(1)


def _gather_body(uidx_hbm, iidx_hbm, utab1, itab1, out,
                 idx_v, qcall, dst, gsem0, gsem1, wsem0, wsem1):
    c = lax.axis_index("c")
    s = lax.axis_index("s")
    wid = s * _NC + c
    base = wid * _BPW
    gsem = (gsem0, gsem1)
    wsem = (wsem0, wsem1)

    pltpu.sync_copy(uidx_hbm.at[pl.ds(base, _BPW)], idx_v)
    _prep_idx(idx_v, qcall)
    _gather_one_table(utab1, qcall, dst, out, 0, wid, gsem, wsem)

    pltpu.sync_copy(iidx_hbm.at[pl.ds(base, _BPW)], idx_v)
    _prep_idx(idx_v, qcall)
    _gather_one_table(itab1, qcall, dst, out, EMBED_DIM, wid, gsem, wsem)


@jax.jit
def _sc_gather(uidx, iidx, utab1, itab1):
    mesh = plsc.VectorSubcoreMesh(core_axis_name="c", subcore_axis_name="s")
    f = functools.partial(
        pl.kernel,
        mesh=mesh,
        out_type=jax.ShapeDtypeStruct((2 * EMBED_DIM, BATCH), jnp.float32),
        scratch_types=[
            pltpu.VMEM((_BPW,), jnp.int32),
            pltpu.VMEM((EMBED_DIM, _BPW), jnp.int32),
            pltpu.VMEM((2 * EMBED_DIM * _CHUNK,), jnp.float32),
            pltpu.SemaphoreType.DMA,
            pltpu.SemaphoreType.DMA,
            pltpu.SemaphoreType.DMA,
            pltpu.SemaphoreType.DMA,
        ],
        compiler_params=pltpu.CompilerParams(use_tc_tiling_on_sc=False),
    )(_gather_body)
    return f(uidx, iidx, utab1, itab1)


_BB = 1024
_NB = BATCH // _BB


def _mlp_body(x_ref, w1u_ref, w1i_ref, b1_ref, w2_ref, b2_ref,
              w3_ref, b3_ref, out_ref):
    x_u = x_ref[pl.ds(0, EMBED_DIM), :]
    x_i = x_ref[pl.ds(EMBED_DIM, EMBED_DIM), :]
    h1 = jnp.dot(w1u_ref[...], x_u, preferred_element_type=jnp.float32)
    h1 = h1 + jnp.dot(w1i_ref[...], x_i, preferred_element_type=jnp.float32)
    h1 = jnp.maximum(h1 + b1_ref[...], 0.0)
    h2 = jnp.dot(w2_ref[...], h1, preferred_element_type=jnp.float32)
    h2 = jnp.maximum(h2 + b2_ref[...], 0.0)
    o = jnp.sum(h2 * w3_ref[...], axis=0) + b3_ref[0, 0]
    out_ref[0, :] = o


@jax.jit
def _tc_mlp(xT, w1u, w1i, b1, w2, b2, w3, b3):
    out2d = pl.pallas_call(
        _mlp_body,
        grid=(_NB,),
        in_specs=[
            pl.BlockSpec((2 * EMBED_DIM, _BB), lambda b: (0, b)),
            pl.BlockSpec((64, EMBED_DIM), lambda b: (0, 0)),
            pl.BlockSpec((64, EMBED_DIM), lambda b: (0, 0)),
            pl.BlockSpec((64, 1), lambda b: (0, 0)),
            pl.BlockSpec((32, 64), lambda b: (0, 0)),
            pl.BlockSpec((32, 1), lambda b: (0, 0)),
            pl.BlockSpec((32, 1), lambda b: (0, 0)),
            pl.BlockSpec((1, 1), lambda b: (0, 0)),
        ],
        out_specs=pl.BlockSpec((1, _BB), lambda b: (0, b)),
        out_shape=jax.ShapeDtypeStruct((1, BATCH), jnp.float32),
    )(xT, w1u, w1i, b1, w2, b2, w3, b3)
    return out2d.reshape(BATCH)


def kernel(user, item, user_table, item_table, W1, b1, W2, b2, W3, b3):
    uidx = user.astype(jnp.int32)
    iidx = item.astype(jnp.int32)
    nrow = user_table.shape[0]
    utab1 = jnp.pad(user_table.T, ((0, 0), (0, _PADR - nrow))).reshape(-1)
    itab1 = jnp.pad(item_table.T, ((0, 0), (0, _PADR - nrow))).reshape(-1)
    xT = _sc_gather(uidx, iidx, utab1, itab1)
    return _tc_mlp(xT, W1[:, :EMBED_DIM], W1[:, EMBED_DIM:],
                   b1.reshape(64, 1), W2, b2.reshape(32, 1),
                   W3.reshape(32, 1), b3.reshape(1, 1))


# speedup vs baseline: 6.6461x; 5.7917x over previous
import functools

import jax
import jax.numpy as jnp
from jax import lax
from jax.experimental import pallas as pl
from jax.experimental.pallas import tpu as pltpu
from jax.experimental.pallas import tpu_sc as plsc

BATCH = 16384
EMBED_DIM = 32

_NC = 2
_NS = 16
_NW = _NC * _NS
_BPW = BATCH // _NW
_CHUNK = 128
_NCHUNK = _BPW // _CHUNK


def _gather_body(uidx_hbm, iidx_hbm, utab, itab, out,
                 uidx_v, iidx_v, urows, irows, sem):
    c = lax.axis_index("c")
    s = lax.axis_index("s")
    wid = s * _NC + c
    base = wid * _BPW
    pltpu.sync_copy(uidx_hbm.at[wid], uidx_v)
    pltpu.sync_copy(iidx_hbm.at[wid], iidx_v)
    copies = []
    for j in range(_NCHUNK):
        copies.append(pltpu.async_copy(
            utab.at[uidx_v.at[j]], urows.at[pl.ds(j * _CHUNK, _CHUNK)], sem))
        copies.append(pltpu.async_copy(
            itab.at[iidx_v.at[j]], irows.at[pl.ds(j * _CHUNK, _CHUNK)], sem))
    for cp in copies:
        cp.wait()
    pltpu.sync_copy(urows, out.at[pl.ds(base, _BPW), pl.ds(0, EMBED_DIM)])
    pltpu.sync_copy(irows, out.at[pl.ds(base, _BPW),
                                  pl.ds(EMBED_DIM, EMBED_DIM)])


@jax.jit
def _sc_gather(uidx3, iidx3, utab, itab):
    mesh = plsc.VectorSubcoreMesh(core_axis_name="c", subcore_axis_name="s")
    f = functools.partial(
        pl.kernel,
        mesh=mesh,
        out_type=jax.ShapeDtypeStruct((BATCH, 128), jnp.float32),
        scratch_types=[
            pltpu.VMEM((_NCHUNK, _CHUNK), jnp.int32),
            pltpu.VMEM((_NCHUNK, _CHUNK), jnp.int32),
            pltpu.VMEM((_BPW, EMBED_DIM), jnp.float32),
            pltpu.VMEM((_BPW, EMBED_DIM), jnp.float32),
            pltpu.SemaphoreType.DMA,
        ],
        compiler_params=pltpu.CompilerParams(use_tc_tiling_on_sc=False),
    )(_gather_body)
    return f(uidx3, iidx3, utab, itab)


_BB = 1024
_NB = BATCH // _BB


def _mlp_body(x_ref, w1u_ref, w1i_ref, b1_ref, w2_ref, b2_ref,
              w3_ref, b3_ref, out_ref):
    u = x_ref[:, pl.ds(0, EMBED_DIM)]
    i = x_ref[:, pl.ds(EMBED_DIM, EMBED_DIM)]
    h1 = jnp.dot(u, w1u_ref[...], preferred_element_type=jnp.float32)
    h1 = h1 + jnp.dot(i, w1i_ref[...],
                      preferred_element_type=jnp.float32)
    h1 = jnp.maximum(h1 + b1_ref[...], 0.0)
    h2 = jnp.dot(h1, w2_ref[...], preferred_element_type=jnp.float32)
    h2 = jnp.maximum(h2 + b2_ref[...], 0.0)
    o = jnp.sum(h2 * w3_ref[...], axis=1) + b3_ref[0, 0]
    out_ref[:, 0] = o


@jax.jit
def _tc_mlp(x, w1u, w1i, b1, w2, b2, w3, b3):
    out2d = pl.pallas_call(
        _mlp_body,
        grid=(_NB,),
        in_specs=[
            pl.BlockSpec((_BB, 128), lambda b: (b, 0)),
            pl.BlockSpec((EMBED_DIM, 64), lambda b: (0, 0)),
            pl.BlockSpec((EMBED_DIM, 64), lambda b: (0, 0)),
            pl.BlockSpec((1, 64), lambda b: (0, 0)),
            pl.BlockSpec((64, 32), lambda b: (0, 0)),
            pl.BlockSpec((1, 32), lambda b: (0, 0)),
            pl.BlockSpec((1, 32), lambda b: (0, 0)),
            pl.BlockSpec((1, 1), lambda b: (0, 0)),
        ],
        out_specs=pl.BlockSpec((_BB, 1), lambda b: (b, 0)),
        out_shape=jax.ShapeDtypeStruct((BATCH, 1), jnp.float32),
    )(x, w1u, w1i, b1, w2, b2, w3, b3)
    return out2d.reshape(BATCH)


def kernel(user, item, user_table, item_table, W1, b1, W2, b2, W3, b3):
    uidx3 = user.astype(jnp.int32).reshape(_NW, _NCHUNK, _CHUNK)
    iidx3 = item.astype(jnp.int32).reshape(_NW, _NCHUNK, _CHUNK)
    x_rows = _sc_gather(uidx3, iidx3, user_table, item_table)
    w1u = W1[:, :EMBED_DIM].T
    w1i = W1[:, EMBED_DIM:].T
    return _tc_mlp(x_rows, w1u, w1i, b1.reshape(1, 64),
                   W2.T, b2.reshape(1, 32), W3, b3.reshape(1, 1))
